# baseline (speedup 1.0000x reference)
import functools

import jax
import jax.numpy as jnp
from jax.experimental import pallas as pl
from jax.experimental.pallas import tpu as pltpu

_B, _C, _N, _K = 8, 384, 256, 9
_NE = _N * _K
_ROWS = 768
_NB = _NE // _ROWS


def _seq_norm2(xsq):
    acc = xsq[0:1, :]
    for c in range(1, _C):
        acc = xsq[c:c + 1, :] + acc
    return acc


def _mod8_tree_sum(xsq):
    chains = [xsq[r:r + 1, :] for r in range(8)]
    for c in range(8, _C):
        r = c % 8
        chains[r] = xsq[c:c + 1, :] + chains[r]
    cur = chains
    for rot in (4, 2, 1):
        cur = [cur[r] + cur[(r + rot) % 8] for r in range(8)]
    return cur[0]


def _core_kernel(x_ref, nn_ref, anorm_ref, gfeat_ref):
    xr = x_ref[0]
    norm = jnp.sqrt(_seq_norm2(xr * xr))
    xn = xr / jnp.maximum(norm, 1e-12)

    col_iota = jax.lax.broadcasted_iota(jnp.int32, (_N, _N), 1)
    row_iota = jax.lax.broadcasted_iota(jnp.int32, (_N, _N), 0)
    eye = (col_iota == row_iota).astype(jnp.float32)

    inner = jax.lax.dot_general(
        xn, xn, (((0,), (0,)), ((), ())),
        preferred_element_type=jnp.float32)
    sq_row = _mod8_tree_sum(xn * xn)
    sq_col = jax.lax.dot_general(
        eye, sq_row, (((1,), (1,)), ((), ())),
        preferred_element_type=jnp.float32,
        precision=jax.lax.Precision.HIGHEST)
    dist = (sq_col + (-2.0 * inner)) + sq_row
    colk = jax.lax.broadcasted_iota(jnp.int32, (_N, _K), 1)

    work = dist
    aacc = jnp.zeros((_N, _N), jnp.float32)
    nn_mat = jnp.zeros((_N, _K), jnp.int32)
    gfeat = jnp.zeros((_N, _K), jnp.float32)
    for j in range(_K):
        mval = jnp.min(work, axis=1, keepdims=True)
        cand = jnp.where(work == mval, col_iota, _N)
        idx = jnp.min(cand, axis=1, keepdims=True)
        oh = col_iota == idx
        nn_mat = nn_mat + jnp.where(colk == j, idx, 0)
        gval = jnp.sum(jnp.where(oh, dist, 0.0), axis=1, keepdims=True)
        gfeat = gfeat + jnp.where(colk == j, gval, 0.0)
        aacc = aacc + oh.astype(jnp.float32)
        work = jnp.where(oh, jnp.inf, work)
    nn_ref[0] = nn_mat
    gfeat_ref[0] = gfeat

    at = jax.lax.dot_general(
        aacc, eye, (((0,), (0,)), ((), ())),
        preferred_element_type=jnp.float32)
    asym = jnp.minimum(aacc + at, 1.0) + eye
    rowsum = jnp.sum(asym, axis=1, keepdims=True)
    rinv = jnp.where(rowsum != 0.0, 1.0 / rowsum, 0.0)
    anorm_ref[0] = asym * rinv


def _edge_kernel(dstr_ref, dstc_ref, e_ref, t_ref, mt_ref):
    rb = pl.program_id(1)
    tcol = dstr_ref[0]
    trow = dstc_ref[0]

    @pl.when(rb == 0)
    def _():
        v = jax.lax.broadcasted_iota(jnp.int32, (_N, _NE), 0)
        se = jax.lax.broadcasted_iota(jnp.int32, (_N, _NE), 1) // _K
        hs = v == se
        ht = v == tcol
        t_ref[0] = hs.astype(jnp.float32) + ht.astype(jnp.float32)
        mt_ref[...] = (hs | ht).astype(jnp.bfloat16)

    row_e = rb * _ROWS + jax.lax.broadcasted_iota(jnp.int32, (_ROWS, 1), 0)
    srow = row_e // _K
    vcol = jax.lax.broadcasted_iota(jnp.int32, (_ROWS, _N), 1)
    mrow = ((vcol == srow) | (vcol == trow)).astype(jnp.bfloat16)
    cnt = jax.lax.dot_general(
        mrow, mt_ref[...], (((1,), (0,)), ((), ())),
        preferred_element_type=jnp.float32)
    bf = jnp.minimum(cnt, 1.0)
    ers = jnp.sum(bf, axis=1, keepdims=True) + 1.0
    einv = jnp.where(ers != 0.0, 1.0 / ers, 0.0)
    e_ref[0] = bf * einv
    rr = jax.lax.broadcasted_iota(jnp.int32, (_ROWS, _ROWS), 0)
    cc = jax.lax.broadcasted_iota(jnp.int32, (_ROWS, _ROWS), 1)
    eyeb = (rr == cc).astype(jnp.float32)
    cnt_sub = jax.lax.dot_general(
        mrow, mt_ref[:, pl.ds(pl.multiple_of(rb * _ROWS, 128), _ROWS)], (((1,), (0,)), ((), ())),
        preferred_element_type=jnp.float32)
    e_ref[0, :, pl.ds(pl.multiple_of(rb * _ROWS, 128), _ROWS)] = (
        jnp.minimum(cnt_sub, 1.0) + eyeb) * einv


@functools.partial(jax.jit)
def kernel(x):
    xr = x.reshape(_B, _C, _N)
    nn, anorm, gfeat = pl.pallas_call(
        _core_kernel,
        grid=(_B,),
        in_specs=[pl.BlockSpec((1, _C, _N), lambda b: (b, 0, 0))],
        out_specs=[
            pl.BlockSpec((1, _N, _K), lambda b: (b, 0, 0)),
            pl.BlockSpec((1, _N, _N), lambda b: (b, 0, 0)),
            pl.BlockSpec((1, _N, _K), lambda b: (b, 0, 0)),
        ],
        out_shape=[
            jax.ShapeDtypeStruct((_B, _N, _K), jnp.int32),
            jax.ShapeDtypeStruct((_B, _N, _N), jnp.float32),
            jax.ShapeDtypeStruct((_B, _N, _K), jnp.float32),
        ],
        compiler_params=pltpu.CompilerParams(
            dimension_semantics=("parallel",)),
    )(xr)

    flat_dst = nn.reshape(_B, _NE)
    dst_row = flat_dst.reshape(_B, 1, _NE)
    dst_col = flat_dst.reshape(_B, _NE, 1)

    e_norm, t_mat = pl.pallas_call(
        _edge_kernel,
        grid=(_B, _NB),
        in_specs=[
            pl.BlockSpec((1, 1, _NE), lambda b, rb: (b, 0, 0)),
            pl.BlockSpec((1, _ROWS, 1), lambda b, rb: (b, rb, 0)),
        ],
        out_specs=[
            pl.BlockSpec((1, _ROWS, _NE), lambda b, rb: (b, rb, 0)),
            pl.BlockSpec((1, _N, _NE), lambda b, rb: (b, 0, 0)),
        ],
        out_shape=[
            jax.ShapeDtypeStruct((_B, _NE, _NE), jnp.float32),
            jax.ShapeDtypeStruct((_B, _N, _NE), jnp.float32),
        ],
        scratch_shapes=[pltpu.VMEM((_N, _NE), jnp.bfloat16)],
        compiler_params=pltpu.CompilerParams(
            dimension_semantics=("parallel", "arbitrary")),
    )(dst_row, dst_col)

    center = jnp.broadcast_to(
        jnp.arange(_N, dtype=nn.dtype)[None, :, None], (_B, _N, _K))
    edge_index = jnp.stack([nn, center], axis=0)
    flat_src = jnp.broadcast_to(
        (jnp.arange(_NE, dtype=nn.dtype) // _K)[None, :], (_B, _NE))
    edges = jnp.stack([flat_src, flat_dst], axis=-1)
    edge_feat = gfeat.reshape(_B, _NE, 1)
    return edge_index, anorm, e_norm, edges, t_mat, edge_feat


# speedup vs baseline: 1.0122x; 1.0122x over previous
import functools

import jax
import jax.numpy as jnp
from jax.experimental import pallas as pl
from jax.experimental.pallas import tpu as pltpu

_B, _C, _N, _K = 8, 384, 256, 9
_NE = _N * _K
_ROWS = 576
_NB = _NE // _ROWS


def _seq_norm2(xsq):
    acc = xsq[0:1, :]
    for c in range(1, _C):
        acc = xsq[c:c + 1, :] + acc
    return acc


def _mod8_tree_sum(xsq):
    chains = [xsq[r:r + 1, :] for r in range(8)]
    for c in range(8, _C):
        r = c % 8
        chains[r] = xsq[c:c + 1, :] + chains[r]
    cur = chains
    for rot in (4, 2, 1):
        cur = [cur[r] + cur[(r + rot) % 8] for r in range(8)]
    return cur[0]


def _core_kernel(x_ref, nn_ref, anorm_ref, gfeat_ref):
    xr = x_ref[0]
    norm = jnp.sqrt(_seq_norm2(xr * xr))
    xn = xr / jnp.maximum(norm, 1e-12)

    col_iota = jax.lax.broadcasted_iota(jnp.int32, (_N, _N), 1)
    row_iota = jax.lax.broadcasted_iota(jnp.int32, (_N, _N), 0)
    eye = (col_iota == row_iota).astype(jnp.float32)

    inner = jax.lax.dot_general(
        xn, xn, (((0,), (0,)), ((), ())),
        preferred_element_type=jnp.float32)
    sq_row = _mod8_tree_sum(xn * xn)
    sq_col = jax.lax.dot_general(
        eye, sq_row, (((1,), (1,)), ((), ())),
        preferred_element_type=jnp.float32,
        precision=jax.lax.Precision.HIGHEST)
    dist = (sq_col + (-2.0 * inner)) + sq_row
    colk = jax.lax.broadcasted_iota(jnp.int32, (_N, _K), 1)

    work = dist
    aacc = jnp.zeros((_N, _N), jnp.float32)
    nn_mat = jnp.zeros((_N, _K), jnp.int32)
    gfeat = jnp.zeros((_N, _K), jnp.float32)
    for j in range(_K):
        mval = jnp.min(work, axis=1, keepdims=True)
        cand = jnp.where(work == mval, col_iota, _N)
        idx = jnp.min(cand, axis=1, keepdims=True)
        oh = col_iota == idx
        nn_mat = nn_mat + jnp.where(colk == j, idx, 0)
        gval = jnp.sum(jnp.where(oh, dist, 0.0), axis=1, keepdims=True)
        gfeat = gfeat + jnp.where(colk == j, gval, 0.0)
        aacc = aacc + oh.astype(jnp.float32)
        work = jnp.where(oh, jnp.inf, work)
    nn_ref[0] = nn_mat
    gfeat_ref[0] = gfeat

    at = jax.lax.dot_general(
        aacc, eye, (((0,), (0,)), ((), ())),
        preferred_element_type=jnp.float32)
    asym = jnp.minimum(aacc + at, 1.0) + eye
    rowsum = jnp.sum(asym, axis=1, keepdims=True)
    rinv = jnp.where(rowsum != 0.0, 1.0 / rowsum, 0.0)
    anorm_ref[0] = asym * rinv


def _edge_kernel(dstr_ref, dstc_ref, e_ref, t_ref, mt_ref):
    rb = pl.program_id(1)
    tcol = dstr_ref[0]
    trow = dstc_ref[0]

    @pl.when(rb == 0)
    def _():
        v = jax.lax.broadcasted_iota(jnp.int32, (_N, _NE), 0)
        se = jax.lax.broadcasted_iota(jnp.int32, (_N, _NE), 1) // _K
        hs = v == se
        ht = v == tcol
        t_ref[0] = hs.astype(jnp.float32) + ht.astype(jnp.float32)
        mt_ref[...] = (hs | ht).astype(jnp.bfloat16)

    row_e = rb * _ROWS + jax.lax.broadcasted_iota(jnp.int32, (_ROWS, 1), 0)
    srow = row_e // _K
    vcol = jax.lax.broadcasted_iota(jnp.int32, (_ROWS, _N), 1)
    mrow = ((vcol == srow) | (vcol == trow)).astype(jnp.bfloat16)
    cnt = jax.lax.dot_general(
        mrow, mt_ref[...], (((1,), (0,)), ((), ())),
        preferred_element_type=jnp.float32)
    col_e = jax.lax.broadcasted_iota(jnp.int32, (1, _NE), 1)
    e = jnp.minimum(cnt, 1.0) + (row_e == col_e).astype(jnp.float32)
    ers = jnp.sum(e, axis=1, keepdims=True)
    einv = jnp.where(ers != 0.0, 1.0 / ers, 0.0)
    e_ref[0] = e * einv


@functools.partial(jax.jit)
def kernel(x):
    xr = x.reshape(_B, _C, _N)
    nn, anorm, gfeat = pl.pallas_call(
        _core_kernel,
        grid=(_B,),
        in_specs=[pl.BlockSpec((1, _C, _N), lambda b: (b, 0, 0))],
        out_specs=[
            pl.BlockSpec((1, _N, _K), lambda b: (b, 0, 0)),
            pl.BlockSpec((1, _N, _N), lambda b: (b, 0, 0)),
            pl.BlockSpec((1, _N, _K), lambda b: (b, 0, 0)),
        ],
        out_shape=[
            jax.ShapeDtypeStruct((_B, _N, _K), jnp.int32),
            jax.ShapeDtypeStruct((_B, _N, _N), jnp.float32),
            jax.ShapeDtypeStruct((_B, _N, _K), jnp.float32),
        ],
        compiler_params=pltpu.CompilerParams(
            dimension_semantics=("parallel",)),
    )(xr)

    flat_dst = nn.reshape(_B, _NE)
    dst_row = flat_dst.reshape(_B, 1, _NE)
    dst_col = flat_dst.reshape(_B, _NE, 1)

    e_norm, t_mat = pl.pallas_call(
        _edge_kernel,
        grid=(_B, _NB),
        in_specs=[
            pl.BlockSpec((1, 1, _NE), lambda b, rb: (b, 0, 0)),
            pl.BlockSpec((1, _ROWS, 1), lambda b, rb: (b, rb, 0)),
        ],
        out_specs=[
            pl.BlockSpec((1, _ROWS, _NE), lambda b, rb: (b, rb, 0)),
            pl.BlockSpec((1, _N, _NE), lambda b, rb: (b, 0, 0)),
        ],
        out_shape=[
            jax.ShapeDtypeStruct((_B, _NE, _NE), jnp.float32),
            jax.ShapeDtypeStruct((_B, _N, _NE), jnp.float32),
        ],
        scratch_shapes=[pltpu.VMEM((_N, _NE), jnp.bfloat16)],
        compiler_params=pltpu.CompilerParams(
            dimension_semantics=("parallel", "arbitrary")),
    )(dst_row, dst_col)

    center = jnp.broadcast_to(
        jnp.arange(_N, dtype=nn.dtype)[None, :, None], (_B, _N, _K))
    edge_index = jnp.stack([nn, center], axis=0)
    flat_src = jnp.broadcast_to(
        (jnp.arange(_NE, dtype=nn.dtype) // _K)[None, :], (_B, _NE))
    edges = jnp.stack([flat_src, flat_dst], axis=-1)
    edge_feat = gfeat.reshape(_B, _NE, 1)
    return edge_index, anorm, e_norm, edges, t_mat, edge_feat


# speedup vs baseline: 1.0405x; 1.0279x over previous
import functools

import jax
import jax.numpy as jnp
from jax.experimental import pallas as pl
from jax.experimental.pallas import tpu as pltpu

_B, _C, _N, _K = 8, 384, 256, 9
_NE = _N * _K
_ROWS = 576
_NB = _NE // _ROWS


def _seq_norm2(xsq):
    acc = xsq[0:1, :]
    for c in range(1, _C):
        acc = xsq[c:c + 1, :] + acc
    return acc


def _mod8_tree_sum(xsq):
    chains = [xsq[r:r + 1, :] for r in range(8)]
    for c in range(8, _C):
        r = c % 8
        chains[r] = xsq[c:c + 1, :] + chains[r]
    cur = chains
    for rot in (4, 2, 1):
        cur = [cur[r] + cur[(r + rot) % 8] for r in range(8)]
    return cur[0]


def _core_kernel(x_ref, nn_ref, anorm_ref, gfeat_ref):
    for s in range(2):
        _core_one(x_ref, nn_ref, anorm_ref, gfeat_ref, s)


def _core_one(x_ref, nn_ref, anorm_ref, gfeat_ref, s):
    xr = x_ref[s]
    norm = jnp.sqrt(_seq_norm2(xr * xr))
    xn = xr / jnp.maximum(norm, 1e-12)

    col_iota = jax.lax.broadcasted_iota(jnp.int32, (_N, _N), 1)
    row_iota = jax.lax.broadcasted_iota(jnp.int32, (_N, _N), 0)
    eye = (col_iota == row_iota).astype(jnp.float32)

    inner = jax.lax.dot_general(
        xn, xn, (((0,), (0,)), ((), ())),
        preferred_element_type=jnp.float32)
    sq_row = _mod8_tree_sum(xn * xn)
    sq_col = jax.lax.dot_general(
        eye, sq_row, (((1,), (1,)), ((), ())),
        preferred_element_type=jnp.float32,
        precision=jax.lax.Precision.HIGHEST)
    dist = (sq_col + (-2.0 * inner)) + sq_row
    colk = jax.lax.broadcasted_iota(jnp.int32, (_N, _K), 1)

    work = dist
    aacc = jnp.zeros((_N, _N), jnp.float32)
    nn_mat = jnp.zeros((_N, _K), jnp.int32)
    gfeat = jnp.zeros((_N, _K), jnp.float32)
    for j in range(_K):
        mval = jnp.min(work, axis=1, keepdims=True)
        cand = jnp.where(work == mval, col_iota, _N)
        idx = jnp.min(cand, axis=1, keepdims=True)
        oh = col_iota == idx
        nn_mat = nn_mat + jnp.where(colk == j, idx, 0)
        gval = jnp.sum(jnp.where(oh, dist, 0.0), axis=1, keepdims=True)
        gfeat = gfeat + jnp.where(colk == j, gval, 0.0)
        aacc = aacc + oh.astype(jnp.float32)
        work = jnp.where(oh, jnp.inf, work)
    nn_ref[s] = nn_mat
    gfeat_ref[s] = gfeat

    at = jax.lax.dot_general(
        aacc, eye, (((0,), (0,)), ((), ())),
        preferred_element_type=jnp.float32)
    asym = jnp.minimum(aacc + at, 1.0) + eye
    rowsum = jnp.sum(asym, axis=1, keepdims=True)
    rinv = jnp.where(rowsum != 0.0, 1.0 / rowsum, 0.0)
    anorm_ref[s] = asym * rinv


def _edge_kernel(dstr_ref, dstc_ref, e_ref, t_ref, mt_ref):
    rb = pl.program_id(1)
    tcol = dstr_ref[0]
    trow = dstc_ref[0]

    @pl.when(rb == 0)
    def _():
        v = jax.lax.broadcasted_iota(jnp.int32, (_N, _NE), 0)
        se = jax.lax.broadcasted_iota(jnp.int32, (_N, _NE), 1) // _K
        hs = v == se
        ht = v == tcol
        t_ref[0] = hs.astype(jnp.float32) + ht.astype(jnp.float32)
        mt_ref[...] = (hs | ht).astype(jnp.bfloat16)

    row_e = rb * _ROWS + jax.lax.broadcasted_iota(jnp.int32, (_ROWS, 1), 0)
    srow = row_e // _K
    vcol = jax.lax.broadcasted_iota(jnp.int32, (_ROWS, _N), 1)
    mrow = ((vcol == srow) | (vcol == trow)).astype(jnp.bfloat16)
    cnt = jax.lax.dot_general(
        mrow, mt_ref[...], (((1,), (0,)), ((), ())),
        preferred_element_type=jnp.float32)
    col_e = jax.lax.broadcasted_iota(jnp.int32, (1, _NE), 1)
    e = jnp.minimum(cnt, 1.0) + (row_e == col_e).astype(jnp.float32)
    ers = jnp.sum(e, axis=1, keepdims=True)
    einv = jnp.where(ers != 0.0, 1.0 / ers, 0.0)
    e_ref[0] = e * einv


@functools.partial(jax.jit)
def kernel(x):
    xr = x.reshape(_B, _C, _N)
    nn, anorm, gfeat = pl.pallas_call(
        _core_kernel,
        grid=(_B // 2,),
        in_specs=[pl.BlockSpec((2, _C, _N), lambda b: (b, 0, 0))],
        out_specs=[
            pl.BlockSpec((2, _N, _K), lambda b: (b, 0, 0)),
            pl.BlockSpec((2, _N, _N), lambda b: (b, 0, 0)),
            pl.BlockSpec((2, _N, _K), lambda b: (b, 0, 0)),
        ],
        out_shape=[
            jax.ShapeDtypeStruct((_B, _N, _K), jnp.int32),
            jax.ShapeDtypeStruct((_B, _N, _N), jnp.float32),
            jax.ShapeDtypeStruct((_B, _N, _K), jnp.float32),
        ],
        compiler_params=pltpu.CompilerParams(
            dimension_semantics=("parallel",)),
    )(xr)

    flat_dst = nn.reshape(_B, _NE)
    dst_row = flat_dst.reshape(_B, 1, _NE)
    dst_col = flat_dst.reshape(_B, _NE, 1)

    e_norm, t_mat = pl.pallas_call(
        _edge_kernel,
        grid=(_B, _NB),
        in_specs=[
            pl.BlockSpec((1, 1, _NE), lambda b, rb: (b, 0, 0)),
            pl.BlockSpec((1, _ROWS, 1), lambda b, rb: (b, rb, 0)),
        ],
        out_specs=[
            pl.BlockSpec((1, _ROWS, _NE), lambda b, rb: (b, rb, 0)),
            pl.BlockSpec((1, _N, _NE), lambda b, rb: (b, 0, 0)),
        ],
        out_shape=[
            jax.ShapeDtypeStruct((_B, _NE, _NE), jnp.float32),
            jax.ShapeDtypeStruct((_B, _N, _NE), jnp.float32),
        ],
        scratch_shapes=[pltpu.VMEM((_N, _NE), jnp.bfloat16)],
        compiler_params=pltpu.CompilerParams(
            dimension_semantics=("parallel", "arbitrary")),
    )(dst_row, dst_col)

    center = jnp.broadcast_to(
        jnp.arange(_N, dtype=nn.dtype)[None, :, None], (_B, _N, _K))
    edge_index = jnp.stack([nn, center], axis=0)
    flat_src = jnp.broadcast_to(
        (jnp.arange(_NE, dtype=nn.dtype) // _K)[None, :], (_B, _NE))
    edges = jnp.stack([flat_src, flat_dst], axis=-1)
    edge_feat = gfeat.reshape(_B, _NE, 1)
    return edge_index, anorm, e_norm, edges, t_mat, edge_feat


# speedup vs baseline: 1.0478x; 1.0070x over previous
import functools

import jax
import jax.numpy as jnp
from jax.experimental import pallas as pl
from jax.experimental.pallas import tpu as pltpu

_B, _C, _N, _K = 8, 384, 256, 9
_NE = _N * _K
_ROWS = 576
_NB = _NE // _ROWS


def _seq_norm2(xsq):
    acc = xsq[0:1, :]
    for c in range(1, _C):
        acc = xsq[c:c + 1, :] + acc
    return acc


def _mod8_tree_sum(xsq):
    chains = [xsq[r:r + 1, :] for r in range(8)]
    for c in range(8, _C):
        r = c % 8
        chains[r] = xsq[c:c + 1, :] + chains[r]
    cur = chains
    for rot in (4, 2, 1):
        cur = [cur[r] + cur[(r + rot) % 8] for r in range(8)]
    return cur[0]


def _core_kernel(x_ref, nn_ref, anorm_ref, gfeat_ref):
    for s in range(4):
        _core_one(x_ref, nn_ref, anorm_ref, gfeat_ref, s)


def _core_one(x_ref, nn_ref, anorm_ref, gfeat_ref, s):
    xr = x_ref[s]
    norm = jnp.sqrt(_seq_norm2(xr * xr))
    xn = xr / jnp.maximum(norm, 1e-12)

    col_iota = jax.lax.broadcasted_iota(jnp.int32, (_N, _N), 1)
    row_iota = jax.lax.broadcasted_iota(jnp.int32, (_N, _N), 0)
    eye = (col_iota == row_iota).astype(jnp.float32)

    inner = jax.lax.dot_general(
        xn, xn, (((0,), (0,)), ((), ())),
        preferred_element_type=jnp.float32)
    sq_row = _mod8_tree_sum(xn * xn)
    sq_col = jax.lax.dot_general(
        eye, sq_row, (((1,), (1,)), ((), ())),
        preferred_element_type=jnp.float32,
        precision=jax.lax.Precision.HIGHEST)
    dist = (sq_col + (-2.0 * inner)) + sq_row
    colk = jax.lax.broadcasted_iota(jnp.int32, (_N, _K), 1)

    work = dist
    aacc = jnp.zeros((_N, _N), jnp.float32)
    nn_mat = jnp.zeros((_N, _K), jnp.int32)
    gfeat = jnp.zeros((_N, _K), jnp.float32)
    for j in range(_K):
        mval = jnp.min(work, axis=1, keepdims=True)
        cand = jnp.where(work == mval, col_iota, _N)
        idx = jnp.min(cand, axis=1, keepdims=True)
        oh = col_iota == idx
        nn_mat = nn_mat + jnp.where(colk == j, idx, 0)
        gval = jnp.sum(jnp.where(oh, dist, 0.0), axis=1, keepdims=True)
        gfeat = gfeat + jnp.where(colk == j, gval, 0.0)
        aacc = aacc + oh.astype(jnp.float32)
        work = jnp.where(oh, jnp.inf, work)
    nn_ref[s] = nn_mat
    gfeat_ref[s] = gfeat

    at = jax.lax.dot_general(
        aacc, eye, (((0,), (0,)), ((), ())),
        preferred_element_type=jnp.float32)
    asym = jnp.minimum(aacc + at, 1.0) + eye
    rowsum = jnp.sum(asym, axis=1, keepdims=True)
    rinv = jnp.where(rowsum != 0.0, 1.0 / rowsum, 0.0)
    anorm_ref[s] = asym * rinv


def _edge_kernel(dstr_ref, dstc_ref, e_ref, t_ref, mt_ref):
    rb = pl.program_id(1)
    tcol = dstr_ref[0]
    trow = dstc_ref[0]

    @pl.when(rb == 0)
    def _():
        v = jax.lax.broadcasted_iota(jnp.int32, (_N, _NE), 0)
        se = jax.lax.broadcasted_iota(jnp.int32, (_N, _NE), 1) // _K
        hs = v == se
        ht = v == tcol
        t_ref[0] = hs.astype(jnp.float32) + ht.astype(jnp.float32)
        mt_ref[...] = (hs | ht).astype(jnp.bfloat16)

    row_e = rb * _ROWS + jax.lax.broadcasted_iota(jnp.int32, (_ROWS, 1), 0)
    srow = row_e // _K
    vcol = jax.lax.broadcasted_iota(jnp.int32, (_ROWS, _N), 1)
    mrow = ((vcol == srow) | (vcol == trow)).astype(jnp.bfloat16)
    cnt = jax.lax.dot_general(
        mrow, mt_ref[...], (((1,), (0,)), ((), ())),
        preferred_element_type=jnp.float32)
    col_e = jax.lax.broadcasted_iota(jnp.int32, (1, _NE), 1)
    e = jnp.minimum(cnt, 1.0) + (row_e == col_e).astype(jnp.float32)
    ers = jnp.sum(e, axis=1, keepdims=True)
    einv = jnp.where(ers != 0.0, 1.0 / ers, 0.0)
    e_ref[0] = e * einv


@functools.partial(jax.jit)
def kernel(x):
    xr = x.reshape(_B, _C, _N)
    nn, anorm, gfeat = pl.pallas_call(
        _core_kernel,
        grid=(_B // 4,),
        in_specs=[pl.BlockSpec((4, _C, _N), lambda b: (b, 0, 0))],
        out_specs=[
            pl.BlockSpec((4, _N, _K), lambda b: (b, 0, 0)),
            pl.BlockSpec((4, _N, _N), lambda b: (b, 0, 0)),
            pl.BlockSpec((4, _N, _K), lambda b: (b, 0, 0)),
        ],
        out_shape=[
            jax.ShapeDtypeStruct((_B, _N, _K), jnp.int32),
            jax.ShapeDtypeStruct((_B, _N, _N), jnp.float32),
            jax.ShapeDtypeStruct((_B, _N, _K), jnp.float32),
        ],
        compiler_params=pltpu.CompilerParams(
            dimension_semantics=("parallel",)),
    )(xr)

    flat_dst = nn.reshape(_B, _NE)
    dst_row = flat_dst.reshape(_B, 1, _NE)
    dst_col = flat_dst.reshape(_B, _NE, 1)

    e_norm, t_mat = pl.pallas_call(
        _edge_kernel,
        grid=(_B, _NB),
        in_specs=[
            pl.BlockSpec((1, 1, _NE), lambda b, rb: (b, 0, 0)),
            pl.BlockSpec((1, _ROWS, 1), lambda b, rb: (b, rb, 0)),
        ],
        out_specs=[
            pl.BlockSpec((1, _ROWS, _NE), lambda b, rb: (b, rb, 0)),
            pl.BlockSpec((1, _N, _NE), lambda b, rb: (b, 0, 0)),
        ],
        out_shape=[
            jax.ShapeDtypeStruct((_B, _NE, _NE), jnp.float32),
            jax.ShapeDtypeStruct((_B, _N, _NE), jnp.float32),
        ],
        scratch_shapes=[pltpu.VMEM((_N, _NE), jnp.bfloat16)],
        compiler_params=pltpu.CompilerParams(
            dimension_semantics=("parallel", "arbitrary")),
    )(dst_row, dst_col)

    center = jnp.broadcast_to(
        jnp.arange(_N, dtype=nn.dtype)[None, :, None], (_B, _N, _K))
    edge_index = jnp.stack([nn, center], axis=0)
    flat_src = jnp.broadcast_to(
        (jnp.arange(_NE, dtype=nn.dtype) // _K)[None, :], (_B, _NE))
    edges = jnp.stack([flat_src, flat_dst], axis=-1)
    edge_feat = gfeat.reshape(_B, _NE, 1)
    return edge_index, anorm, e_norm, edges, t_mat, edge_feat
